# baseline (speedup 1.0000x reference)
import jax
import jax.numpy as jnp
from jax import lax
from jax.experimental import pallas as pl
from jax.experimental.pallas import tpu as pltpu
from jax.experimental.pallas import tpu_sc as plsc

_NC = 2
_NS = 16
_K = 128
_G = 8
_DH = 64
_WS = 80


def _sc_edge_agg(x0, x1, srcp, dstp, acc_rows, chs):
    rows_pt = acc_rows // _NS
    n = x0.shape[0]
    nst = (n // (_NS * 8)) * 8
    rem = n - nst * _NS

    def body(x0_hbm, x1_hbm, src_hbm, dst_hbm, out_hbm, src_v, dst_v,
             rows_a, rows_b, zbuf, xsp, acc_sh, sem_a, sem_b):
        cid = lax.axis_index("c")
        sid = lax.axis_index("s")

        def stage(xh):
            pltpu.async_copy(xh.at[pl.ds(sid * nst, nst)],
                             xsp.at[pl.ds(sid * nst, nst)], sem_b)
            if rem:
                @pl.when(sid == _NS - 1)
                def _():
                    pltpu.async_copy(xh.at[pl.ds(nst * _NS, rem)],
                                     xsp.at[pl.ds(nst * _NS, rem)], sem_b)

        @pl.when(cid == 0)
        def _():
            stage(x0_hbm)

        @pl.when(cid == 1)
        def _():
            stage(x1_hbm)

        def zrow(r, _):
            for j in range(_DH // 16):
                zbuf[r, pl.ds(j * 16, 16)] = jnp.zeros((16,), jnp.float32)
            return ()
        lax.fori_loop(0, rows_pt // 4, zrow, ())
        for rep in range(4):
            pltpu.sync_copy(
                zbuf,
                acc_sh.at[pl.ds(sid * rows_pt + rep * (rows_pt // 4),
                                rows_pt // 4)])

        pltpu.make_async_copy(x0_hbm.at[pl.ds(sid * nst, nst)],
                              xsp.at[pl.ds(sid * nst, nst)], sem_b).wait()
        if rem:
            @pl.when(sid == _NS - 1)
            def _():
                pltpu.make_async_copy(x0_hbm.at[pl.ds(nst * _NS, rem)],
                                      xsp.at[pl.ds(nst * _NS, rem)],
                                      sem_b).wait()

        plsc.subcore_barrier()

        def win(w, _):
            b = pl.multiple_of(sid * chs + w * _WS, 8)
            pltpu.sync_copy(src_hbm.at[pl.ds(b, _WS)], src_v)
            pltpu.sync_copy(dst_hbm.at[pl.ds(b, _WS)], dst_v)
            pltpu.async_copy(xsp.at[src_v.at[0]], rows_a, sem_a)

            def pair(p, _):
                c0 = 2 * p
                gb = pltpu.async_copy(xsp.at[src_v.at[c0 + 1]], rows_b,
                                      sem_b)
                pltpu.make_async_copy(xsp.at[src_v.at[c0]], rows_a,
                                      sem_a).wait()
                pltpu.sync_copy(rows_a, acc_sh.at[dst_v.at[c0]], add=True)

                @pl.when(c0 + 2 < _WS)
                def _():
                    pltpu.async_copy(xsp.at[src_v.at[c0 + 2]], rows_a,
                                     sem_a)

                gb.wait()
                pltpu.sync_copy(rows_b, acc_sh.at[dst_v.at[c0 + 1]], add=True)
                return ()
            lax.fori_loop(0, _WS // 2, pair, ())
            return ()
        lax.fori_loop(0, chs // _WS, win, ())

        plsc.subcore_barrier()
        pltpu.sync_copy(acc_sh.at[pl.ds(sid * rows_pt, rows_pt)],
                        out_hbm.at[cid, pl.ds(sid * rows_pt, rows_pt)])

    return pl.kernel(
        body,
        out_type=jax.ShapeDtypeStruct((_NC, acc_rows, _DH), jnp.float32),
        mesh=plsc.VectorSubcoreMesh(core_axis_name="c", subcore_axis_name="s",
                                    num_cores=_NC, num_subcores=_NS),
        compiler_params=pltpu.CompilerParams(use_tc_tiling_on_sc=False),
        scratch_types=[
            pltpu.VMEM((_WS, _K), jnp.int32),
            pltpu.VMEM((_WS, _K), jnp.int32),
            pltpu.VMEM((_K, _DH), jnp.float32),
            pltpu.VMEM((_K, _DH), jnp.float32),
            pltpu.VMEM((rows_pt // 4, _DH), jnp.float32),
            pltpu.VMEM_SHARED((n, _DH), jnp.float32),
            pltpu.VMEM_SHARED((acc_rows, _DH), jnp.float32),
            pltpu.SemaphoreType.DMA,
            pltpu.SemaphoreType.DMA,
        ],
    )(x0, x1, srcp, dstp)


def _dot(a, b):
    return lax.dot_general(a, b, (((1,), (0,)), ((), ())),
                           preferred_element_type=jnp.float32)


def _l1_body(x_ref, a0_ref, a1_ref, w_ref, b_ref, sc_ref, sh_ref, e_ref,
             o0_ref, o1_ref):
    agg = jnp.concatenate([a0_ref[0], a1_ref[0]], axis=1)
    h = x_ref[...] * e_ref[0, 0] + agg
    y = jnp.maximum(_dot(h, w_ref[...]) + b_ref[...], 0.0)
    y = jnp.maximum(y * sc_ref[...] + sh_ref[...], 0.0)
    o0_ref[...] = y[:, :_DH]
    o1_ref[...] = y[:, _DH:]


def _l2_body(x0_ref, x1_ref, a0_ref, a1_ref, w_ref, b_ref, sc_ref, sh_ref,
             e_ref, i_ref, wd1_ref, bd1_ref, wd2_ref, bd2_ref, wo_ref,
             bo_ref, o_ref, pool_ref):
    k = pl.program_id(0)
    nb = pl.num_programs(0)

    @pl.when(k == 0)
    def _init():
        pool_ref[...] = jnp.full((_G, 128), -jnp.inf, jnp.float32)

    x = jnp.concatenate([x0_ref[...], x1_ref[...]], axis=1)
    agg = jnp.concatenate([a0_ref[0], a1_ref[0]], axis=1)
    h = x * e_ref[0, 0] + agg
    y = jnp.maximum(_dot(h, w_ref[...]) + b_ref[...], 0.0)
    y = jnp.maximum(y * sc_ref[...] + sh_ref[...], 0.0)

    ib = i_ref[...]
    neg = jnp.float32(-jnp.inf)
    parts = [jnp.max(jnp.where(ib == jnp.int8(g), y, neg), axis=0,
                     keepdims=True)
             for g in range(_G)]
    pool_ref[...] = jnp.maximum(pool_ref[...], jnp.concatenate(parts, axis=0))

    @pl.when(k == nb - 1)
    def _head():
        p = pool_ref[...]
        d1 = jnp.maximum(_dot(p, wd1_ref[...]) + bd1_ref[...], 0.0)
        d2 = jnp.maximum(_dot(d1, wd2_ref[...]) + bd2_ref[...], 0.0)
        lg = _dot(d2, wo_ref[...]) + bo_ref[...]
        m = jnp.max(lg, axis=1, keepdims=True)
        ex = jnp.exp(lg - m)
        o_ref[...] = ex / jnp.sum(ex, axis=1, keepdims=True)


def kernel(x, edge_index, i, eps1, W1, b1, gamma1, beta1, mean1, var1,
           eps2, W2, b2, gamma2, beta2, mean2, var2,
           Wd1, bd1, Wd2, bd2, Wo, bo):
    N, D = x.shape
    E = edge_index.shape[1]
    C = Wo.shape[1]
    blk = 2000
    nb = N // blk
    acc_rows = ((N + 1 + 127) // 128) * 128
    chs = ((-(-E // (_NS * _K)) + _WS - 1) // _WS) * _WS
    cht = _NS * chs
    epad = cht * _K - E

    src = edge_index[0]
    dst = edge_index[1]
    if epad:
        src = jnp.concatenate([src, jnp.zeros((epad,), jnp.int32)])
        dst = jnp.concatenate([dst, jnp.full((epad,), N, jnp.int32)])
    srcp = src.reshape(cht, _K)
    dstp = dst.reshape(cht, _K)

    s1 = (gamma1 * lax.rsqrt(var1 + 1e-3)).reshape(1, D)
    t1 = beta1.reshape(1, D) - mean1.reshape(1, D) * s1
    s2 = (gamma2 * lax.rsqrt(var2 + 1e-3)).reshape(1, D)
    t2 = beta2.reshape(1, D) - mean2.reshape(1, D) * s2
    e1 = (1.0 + eps1).reshape(1, 1)
    e2 = (1.0 + eps2).reshape(1, 1)
    ib = jnp.broadcast_to(i[:, None].astype(jnp.int8), (N, 128))

    bd1p = bd1.reshape(1, 128)
    wd2p = jnp.pad(Wd2, ((0, 0), (0, 128 - Wd2.shape[1])))
    bd2p = jnp.pad(bd2, (0, 128 - bd2.shape[0])).reshape(1, 128)
    wop = jnp.pad(Wo, ((0, 128 - Wo.shape[0]), (0, 128 - C)))
    bop = jnp.pad(bo, (0, 128 - C), constant_values=-1e30).reshape(1, 128)

    x0 = x[:, :_DH]
    x1 = x[:, _DH:]

    row_spec = pl.BlockSpec((blk, 128), lambda k: (k, 0))
    half_spec = pl.BlockSpec((blk, _DH), lambda k: (k, 0))
    agg_spec0 = pl.BlockSpec((1, blk, _DH), lambda k: (0, k, 0))
    agg_spec1 = pl.BlockSpec((1, blk, _DH), lambda k: (1, k, 0))
    full_w = pl.BlockSpec((128, 128), lambda k: (0, 0))
    full_v = pl.BlockSpec((1, 128), lambda k: (0, 0))
    smem_s = pl.BlockSpec(memory_space=pltpu.SMEM)

    agg1 = _sc_edge_agg(x0, x1, srcp, dstp, acc_rows, chs)
    h1a, h1b = pl.pallas_call(
        _l1_body,
        grid=(nb,),
        in_specs=[row_spec, agg_spec0, agg_spec1, full_w, full_v, full_v,
                  full_v, smem_s],
        out_specs=[half_spec, half_spec],
        out_shape=[jax.ShapeDtypeStruct((N, _DH), jnp.float32),
                   jax.ShapeDtypeStruct((N, _DH), jnp.float32)],
    )(x, agg1, agg1, W1, b1.reshape(1, D), s1, t1, e1)

    agg2 = _sc_edge_agg(h1a, h1b, srcp, dstp, acc_rows, chs)
    out = pl.pallas_call(
        _l2_body,
        grid=(nb,),
        in_specs=[half_spec, half_spec, agg_spec0, agg_spec1, full_w,
                  full_v, full_v, full_v, smem_s, row_spec, full_w, full_v,
                  full_w, full_v, full_w, full_v],
        out_specs=pl.BlockSpec((_G, 128), lambda k: (0, 0)),
        out_shape=jax.ShapeDtypeStruct((_G, 128), jnp.float32),
        scratch_shapes=[pltpu.VMEM((_G, 128), jnp.float32)],
        compiler_params=pltpu.CompilerParams(
            dimension_semantics=("arbitrary",)),
    )(h1a, h1b, agg2, agg2, W2, b2.reshape(1, D), s2, t2, e2, ib,
      Wd1, bd1p, wd2p, bd2p, wop, bop)

    return out[:, :C]


# speedup vs baseline: 1.0076x; 1.0076x over previous
import jax
import jax.numpy as jnp
from jax import lax
from jax.experimental import pallas as pl
from jax.experimental.pallas import tpu as pltpu
from jax.experimental.pallas import tpu_sc as plsc

_NC = 2
_NS = 16
_K = 128
_G = 8
_DH = 64
_WS = 80


def _sc_edge_agg(x0, x1, srcp, dstp, acc_rows, chs):
    rows_pt = acc_rows // _NS
    n = x0.shape[0]
    nst = (n // (_NS * 8)) * 8
    rem = n - nst * _NS

    def body(x0_hbm, x1_hbm, src_hbm, dst_hbm, out_hbm, src_v, dst_v,
             rows_a, rows_b, zbuf, xsp, acc_sh, sem_a, sem_b):
        cid = lax.axis_index("c")
        sid = lax.axis_index("s")

        def stage(xh):
            pltpu.async_copy(xh.at[pl.ds(sid * nst, nst)],
                             xsp.at[pl.ds(sid * nst, nst)], sem_b)
            if rem:
                @pl.when(sid == _NS - 1)
                def _():
                    pltpu.async_copy(xh.at[pl.ds(nst * _NS, rem)],
                                     xsp.at[pl.ds(nst * _NS, rem)], sem_b)

        @pl.when(cid == 0)
        def _():
            stage(x0_hbm)

        @pl.when(cid == 1)
        def _():
            stage(x1_hbm)

        def zrow(r, _):
            for j in range(_DH // 16):
                zbuf[r, pl.ds(j * 16, 16)] = jnp.zeros((16,), jnp.float32)
            return ()
        lax.fori_loop(0, rows_pt // 4, zrow, ())
        for rep in range(4):
            pltpu.sync_copy(
                zbuf,
                acc_sh.at[pl.ds(sid * rows_pt + rep * (rows_pt // 4),
                                rows_pt // 4)])

        pltpu.make_async_copy(x0_hbm.at[pl.ds(sid * nst, nst)],
                              xsp.at[pl.ds(sid * nst, nst)], sem_b).wait()
        if rem:
            @pl.when(sid == _NS - 1)
            def _():
                pltpu.make_async_copy(x0_hbm.at[pl.ds(nst * _NS, rem)],
                                      xsp.at[pl.ds(nst * _NS, rem)],
                                      sem_b).wait()

        plsc.subcore_barrier()

        def win(w, _):
            b = pl.multiple_of(sid * chs + w * _WS, 8)
            pltpu.sync_copy(src_hbm.at[pl.ds(b, _WS)], src_v)
            pltpu.sync_copy(dst_hbm.at[pl.ds(b, _WS)], dst_v)
            pltpu.async_copy(xsp.at[src_v.at[0]], rows_a, sem_a)

            def pair(p, _):
                c0 = 2 * p
                gb = pltpu.async_copy(xsp.at[src_v.at[c0 + 1]], rows_b,
                                      sem_b)
                pltpu.make_async_copy(xsp.at[src_v.at[c0]], rows_a,
                                      sem_a).wait()
                pltpu.sync_copy(rows_a, acc_sh.at[dst_v.at[c0]], add=True)

                @pl.when(c0 + 2 < _WS)
                def _():
                    pltpu.async_copy(xsp.at[src_v.at[c0 + 2]], rows_a,
                                     sem_a)

                gb.wait()
                pltpu.sync_copy(rows_b, acc_sh.at[dst_v.at[c0 + 1]], add=True)
                return ()
            lax.fori_loop(0, _WS // 2, pair, ())
            return ()
        lax.fori_loop(0, chs // _WS, win, ())

        plsc.subcore_barrier()
        pltpu.sync_copy(acc_sh.at[pl.ds(sid * rows_pt, rows_pt)],
                        out_hbm.at[cid, pl.ds(sid * rows_pt, rows_pt)])

    return pl.kernel(
        body,
        out_type=jax.ShapeDtypeStruct((_NC, acc_rows, _DH), jnp.float32),
        mesh=plsc.VectorSubcoreMesh(core_axis_name="c", subcore_axis_name="s",
                                    num_cores=_NC, num_subcores=_NS),
        compiler_params=pltpu.CompilerParams(use_tc_tiling_on_sc=False),
        scratch_types=[
            pltpu.VMEM((_WS, _K), jnp.int32),
            pltpu.VMEM((_WS, _K), jnp.int32),
            pltpu.VMEM((_K, _DH), jnp.float32),
            pltpu.VMEM((_K, _DH), jnp.float32),
            pltpu.VMEM((rows_pt // 4, _DH), jnp.float32),
            pltpu.VMEM_SHARED((n, _DH), jnp.float32),
            pltpu.VMEM_SHARED((acc_rows, _DH), jnp.float32),
            pltpu.SemaphoreType.DMA,
            pltpu.SemaphoreType.DMA,
        ],
    )(x0, x1, srcp, dstp)


def _dot(a, b):
    return lax.dot_general(a, b, (((1,), (0,)), ((), ())),
                           preferred_element_type=jnp.float32)


def _l1_body(x_ref, a0_ref, a1_ref, w_ref, b_ref, sc_ref, sh_ref, e_ref,
             o0_ref, o1_ref):
    agg = jnp.concatenate([a0_ref[0], a1_ref[0]], axis=1)
    h = x_ref[...] * e_ref[0, 0] + agg
    y = jnp.maximum(_dot(h, w_ref[...]) + b_ref[...], 0.0)
    y = jnp.maximum(y * sc_ref[...] + sh_ref[...], 0.0)
    o0_ref[...] = y[:, :_DH]
    o1_ref[...] = y[:, _DH:]


def _l2_body(x0_ref, x1_ref, a0_ref, a1_ref, w_ref, b_ref, sc_ref, sh_ref,
             e_ref, i_ref, wd1_ref, bd1_ref, wd2_ref, bd2_ref, wo_ref,
             bo_ref, o_ref, pool_ref):
    k = pl.program_id(0)
    nb = pl.num_programs(0)

    @pl.when(k == 0)
    def _init():
        pool_ref[...] = jnp.full((_G, 128), -jnp.inf, jnp.float32)

    x = jnp.concatenate([x0_ref[...], x1_ref[...]], axis=1)
    agg = jnp.concatenate([a0_ref[0], a1_ref[0]], axis=1)
    h = x * e_ref[0, 0] + agg
    y = jnp.maximum(_dot(h, w_ref[...]) + b_ref[...], 0.0)
    y = jnp.maximum(y * sc_ref[...] + sh_ref[...], 0.0)

    ib = i_ref[...]
    neg = jnp.float32(-jnp.inf)
    parts = [jnp.max(jnp.where(ib == g, y, neg), axis=0, keepdims=True)
             for g in range(_G)]
    pool_ref[...] = jnp.maximum(pool_ref[...], jnp.concatenate(parts, axis=0))

    @pl.when(k == nb - 1)
    def _head():
        p = pool_ref[...]
        d1 = jnp.maximum(_dot(p, wd1_ref[...]) + bd1_ref[...], 0.0)
        d2 = jnp.maximum(_dot(d1, wd2_ref[...]) + bd2_ref[...], 0.0)
        lg = _dot(d2, wo_ref[...]) + bo_ref[...]
        m = jnp.max(lg, axis=1, keepdims=True)
        ex = jnp.exp(lg - m)
        o_ref[...] = ex / jnp.sum(ex, axis=1, keepdims=True)


def kernel(x, edge_index, i, eps1, W1, b1, gamma1, beta1, mean1, var1,
           eps2, W2, b2, gamma2, beta2, mean2, var2,
           Wd1, bd1, Wd2, bd2, Wo, bo):
    N, D = x.shape
    E = edge_index.shape[1]
    C = Wo.shape[1]
    blk = 2000
    nb = N // blk
    acc_rows = ((N + 1 + 127) // 128) * 128
    chs = ((-(-E // (_NS * _K)) + _WS - 1) // _WS) * _WS
    cht = _NS * chs
    epad = cht * _K - E

    src = edge_index[0]
    dst = edge_index[1]
    if epad:
        src = jnp.concatenate([src, jnp.zeros((epad,), jnp.int32)])
        dst = jnp.concatenate([dst, jnp.full((epad,), N, jnp.int32)])
    srcp = src.reshape(cht, _K)
    dstp = dst.reshape(cht, _K)

    s1 = (gamma1 * lax.rsqrt(var1 + 1e-3)).reshape(1, D)
    t1 = beta1.reshape(1, D) - mean1.reshape(1, D) * s1
    s2 = (gamma2 * lax.rsqrt(var2 + 1e-3)).reshape(1, D)
    t2 = beta2.reshape(1, D) - mean2.reshape(1, D) * s2
    e1 = (1.0 + eps1).reshape(1, 1)
    e2 = (1.0 + eps2).reshape(1, 1)
    ib = jnp.broadcast_to(i[:, None], (N, 128))

    bd1p = bd1.reshape(1, 128)
    wd2p = jnp.pad(Wd2, ((0, 0), (0, 128 - Wd2.shape[1])))
    bd2p = jnp.pad(bd2, (0, 128 - bd2.shape[0])).reshape(1, 128)
    wop = jnp.pad(Wo, ((0, 128 - Wo.shape[0]), (0, 128 - C)))
    bop = jnp.pad(bo, (0, 128 - C), constant_values=-1e30).reshape(1, 128)

    x0 = x[:, :_DH]
    x1 = x[:, _DH:]

    row_spec = pl.BlockSpec((blk, 128), lambda k: (k, 0))
    half_spec = pl.BlockSpec((blk, _DH), lambda k: (k, 0))
    agg_spec0 = pl.BlockSpec((1, blk, _DH), lambda k: (0, k, 0))
    agg_spec1 = pl.BlockSpec((1, blk, _DH), lambda k: (1, k, 0))
    full_w = pl.BlockSpec((128, 128), lambda k: (0, 0))
    full_v = pl.BlockSpec((1, 128), lambda k: (0, 0))
    smem_s = pl.BlockSpec(memory_space=pltpu.SMEM)

    agg1 = _sc_edge_agg(x0, x1, srcp, dstp, acc_rows, chs)
    h1a, h1b = pl.pallas_call(
        _l1_body,
        grid=(nb,),
        in_specs=[row_spec, agg_spec0, agg_spec1, full_w, full_v, full_v,
                  full_v, smem_s],
        out_specs=[half_spec, half_spec],
        out_shape=[jax.ShapeDtypeStruct((N, _DH), jnp.float32),
                   jax.ShapeDtypeStruct((N, _DH), jnp.float32)],
    )(x, agg1, agg1, W1, b1.reshape(1, D), s1, t1, e1)

    agg2 = _sc_edge_agg(h1a, h1b, srcp, dstp, acc_rows, chs)
    out = pl.pallas_call(
        _l2_body,
        grid=(nb,),
        in_specs=[half_spec, half_spec, agg_spec0, agg_spec1, full_w,
                  full_v, full_v, full_v, smem_s, row_spec, full_w, full_v,
                  full_w, full_v, full_w, full_v],
        out_specs=pl.BlockSpec((_G, 128), lambda k: (0, 0)),
        out_shape=jax.ShapeDtypeStruct((_G, 128), jnp.float32),
        scratch_shapes=[pltpu.VMEM((_G, 128), jnp.float32)],
        compiler_params=pltpu.CompilerParams(
            dimension_semantics=("arbitrary",)),
    )(h1a, h1b, agg2, agg2, W2, b2.reshape(1, D), s2, t2, e2, ib,
      Wd1, bd1p, wd2p, bd2p, wop, bop)

    return out[:, :C]
